# baseline (speedup 1.0000x reference)
import functools

import jax
import jax.numpy as jnp
from jax.experimental import pallas as pl
from jax.experimental.pallas import tpu as pltpu

P = 128
KNN = 16
KP = KNN * P
EPS = 1e-5
NEG = -3.0e38


def _acc(b, ref, val):
    @pl.when(b == 0)
    def _():
        ref[...] = val

    @pl.when(b > 0)
    def _():
        ref[...] += val


def _bn_coeffs(s_full, q_full, g, bvec, n):
    tot = s_full[0]
    tot2 = q_full[0]
    m = tot / n
    v = tot2 / n - m * m
    scale = g / jnp.sqrt(v + EPS)
    shift = bvec - m * scale
    return scale[:, None], shift[:, None]


def _edge_pass(b, pts, fts, A, Bm, sc_ref, y_ref, sp_ref, qp_ref):
    xx = jnp.sum(pts * pts, axis=0, keepdims=True)
    G = jnp.dot(pts.T, pts, preferred_element_type=jnp.float32)
    pd = 2.0 * G - xx - xx.T
    colidx = jax.lax.broadcasted_iota(jnp.int32, (P, P), 1)
    base = jnp.dot(A, fts, preferred_element_type=jnp.float32)
    Hm = jnp.dot(Bm, fts, preferred_element_type=jnp.float32)
    w = pd
    for j in range(KNN + 1):
        mrow = jnp.max(w, axis=1, keepdims=True)
        cand = jnp.where(w == mrow, colidx, jnp.int32(2 ** 30))
        amin = jnp.min(cand, axis=1, keepdims=True)
        issel = colidx == amin
        w = jnp.where(issel, NEG, w)
        if j > 0:
            sel = issel.astype(jnp.float32)
            sc_ref[:, (j - 1) * P:j * P] = sel.T
    gath = jnp.dot(Hm, sc_ref[...], preferred_element_type=jnp.float32)
    y = gath + jnp.concatenate([base] * KNN, axis=1)
    y_ref[0] = y
    _acc(b, sp_ref, jnp.sum(y, axis=1)[None, :])
    _acc(b, qp_ref, jnp.sum(y * y, axis=1)[None, :])


def _finish_block(y3, s3, q3, gb3, scpre, ssc, qsc, gbsc, mrow, nb, n2):
    sc3, sh3 = _bn_coeffs(s3, q3, gb3[0], gb3[1], n2)
    x = jnp.maximum(sc3 * y3 + sh3, 0.0)
    acc = x[:, :P]
    for j in range(1, KNN):
        acc = acc + x[:, j * P:(j + 1) * P]
    fmean = acc * (1.0 / KNN)
    scs, shs = _bn_coeffs(ssc, qsc, gbsc[0], gbsc[1], nb)
    sc = scs * scpre + shs
    return jnp.maximum(sc + fmean, 0.0) * mrow


def _b1_kernel(pts_ref, f_ref, m_ref, s0_ref, q0_ref, gb0_ref, A_ref,
               Bm_ref, scW_ref, y_ref, sp_ref, qp_ref, scp_ref, ssc_ref,
               qsc_ref, sc_ref, *, nb):
    b = pl.program_id(0)
    mrow = m_ref[0]
    fm = f_ref[0] * mrow
    scale, shift = _bn_coeffs(s0_ref[...], q0_ref[...], gb0_ref[0],
                              gb0_ref[1], nb)
    fts = (scale * fm + shift) * mrow
    pts = pts_ref[0] * mrow + (1.0 - mrow) * 1e9
    scpre = jnp.dot(scW_ref[...], fts, preferred_element_type=jnp.float32)
    scp_ref[0] = scpre
    _acc(b, ssc_ref, jnp.sum(scpre, axis=1)[None, :])
    _acc(b, qsc_ref, jnp.sum(scpre * scpre, axis=1)[None, :])
    _edge_pass(b, pts, fts, A_ref[...], Bm_ref[...], sc_ref, y_ref,
               sp_ref, qp_ref)


def _mid_kernel(y_ref, s_ref, q_ref, gb_ref, W_ref, yo_ref, sp_ref, qp_ref,
                *, n2):
    b = pl.program_id(0)
    scale, shift = _bn_coeffs(s_ref[...], q_ref[...], gb_ref[0], gb_ref[1],
                              n2)
    x = jnp.maximum(scale * y_ref[0] + shift, 0.0)
    y = jnp.dot(W_ref[...], x, preferred_element_type=jnp.float32)
    yo_ref[0] = y
    _acc(b, sp_ref, jnp.sum(y, axis=1)[None, :])
    _acc(b, qp_ref, jnp.sum(y * y, axis=1)[None, :])


def _trans_kernel(y3_ref, s3_ref, q3_ref, gb3_ref, scp_ref, ssc_ref,
                  qsc_ref, gbsc_ref, m_ref, A_ref, Bm_ref, scW_ref,
                  yn_ref, spn_ref, qpn_ref, scpn_ref, sscn_ref, qscn_ref,
                  sc_ref, *, nb, n2):
    b = pl.program_id(0)
    mrow = m_ref[0]
    fts = _finish_block(y3_ref[0], s3_ref[...], q3_ref[...], gb3_ref[...],
                        scp_ref[0], ssc_ref[...], qsc_ref[...],
                        gbsc_ref[...], mrow, nb, n2)
    pts = fts + (1.0 - mrow) * 1e9
    scpre = jnp.dot(scW_ref[...], fts, preferred_element_type=jnp.float32)
    scpn_ref[0] = scpre
    _acc(b, sscn_ref, jnp.sum(scpre, axis=1)[None, :])
    _acc(b, qscn_ref, jnp.sum(scpre * scpre, axis=1)[None, :])
    _edge_pass(b, pts, fts, A_ref[...], Bm_ref[...], sc_ref, yn_ref,
               spn_ref, qpn_ref)


def _final_kernel(y3_ref, s3_ref, q3_ref, gb3_ref, scp_ref, ssc_ref,
                  qsc_ref, gbsc_ref, m_ref, w1_ref, b1_ref, w2_ref, b2_ref,
                  o_ref, *, nb, n2):
    mrow = m_ref[0]
    fts = _finish_block(y3_ref[0], s3_ref[...], q3_ref[...], gb3_ref[...],
                        scp_ref[0], ssc_ref[...], qsc_ref[...],
                        gbsc_ref[...], mrow, nb, n2)
    cnt = jnp.maximum(jnp.sum(mrow), 1.0)
    pooled = (jnp.sum(fts, axis=1) / cnt)[None, :]
    h = jnp.maximum(
        jnp.dot(pooled, w1_ref[...], preferred_element_type=jnp.float32)
        + b1_ref[...], 0.0)
    o_ref[0] = (jnp.dot(h, w2_ref[...], preferred_element_type=jnp.float32)
                + b2_ref[...])


def _stats0_kernel(f_ref, m_ref, s_ref, q_ref):
    b = pl.program_id(0)
    fm = f_ref[0] * m_ref[0]
    _acc(b, s_ref, jnp.sum(fm, axis=1)[None, :])
    _acc(b, q_ref, jnp.sum(fm * fm, axis=1)[None, :])


def _full(shape):
    nd = len(shape)
    return pl.BlockSpec(shape, lambda b: (0,) * nd)


def _perb(shape):
    nd = len(shape)
    return pl.BlockSpec((1,) + shape[1:], lambda b: (b,) + (0,) * (nd - 1))


def kernel(points, features, mask, params):
    B = points.shape[0]
    C0 = features.shape[1]
    nb = float(B * P)
    n2 = float(B * P * KNN)
    f32 = jnp.float32
    sel_scratch = [pltpu.VMEM((P, KP), f32)]

    def call(body, ins, in_specs, outs, out_specs, scratch=None):
        return pl.pallas_call(
            body, grid=(B,), in_specs=in_specs, out_specs=out_specs,
            out_shape=outs, scratch_shapes=scratch or [],
            interpret=False)(*ins)

    s0, q0 = call(
        _stats0_kernel, (features, mask),
        [_perb(features.shape), _perb(mask.shape)],
        (jax.ShapeDtypeStruct((1, C0), f32),
         jax.ShapeDtypeStruct((1, C0), f32)),
        [_full((1, C0)), _full((1, C0))])

    gb0 = jnp.stack([params['bn_fts_g'], params['bn_fts_b']])

    state = None
    cin = C0
    for i, blk in enumerate(params['blocks']):
        W1 = blk['convW'][0]
        A = W1[:, :cin] - W1[:, cin:]
        Bm = W1[:, cin:]
        O1 = A.shape[0]
        outs1 = (jax.ShapeDtypeStruct((B, O1, KP), f32),
                 jax.ShapeDtypeStruct((1, O1), f32),
                 jax.ShapeDtypeStruct((1, O1), f32),
                 jax.ShapeDtypeStruct((B, O1, P), f32),
                 jax.ShapeDtypeStruct((1, O1), f32),
                 jax.ShapeDtypeStruct((1, O1), f32))
        ospec1 = [_perb((B, O1, KP)), _full((1, O1)), _full((1, O1)),
                  _perb((B, O1, P)), _full((1, O1)), _full((1, O1))]
        if i == 0:
            y, sp, qp, scpre, ssc, qsc = call(
                functools.partial(_b1_kernel, nb=nb),
                (points, features, mask, s0, q0, gb0, A, Bm, blk['scW']),
                [_perb(points.shape), _perb(features.shape),
                 _perb(mask.shape), _full((1, C0)), _full((1, C0)),
                 _full(gb0.shape), _full(A.shape), _full(Bm.shape),
                 _full(blk['scW'].shape)],
                outs1, ospec1, sel_scratch)
        else:
            y3, s3, q3, gb3, pscpre, pssc, pqsc, gbsc = state
            Op = y3.shape[1]
            y, sp, qp, scpre, ssc, qsc = call(
                functools.partial(_trans_kernel, nb=nb, n2=n2),
                (y3, s3, q3, gb3, pscpre, pssc, pqsc, gbsc, mask, A, Bm,
                 blk['scW']),
                [_perb(y3.shape), _full((1, Op)), _full((1, Op)),
                 _full(gb3.shape), _perb(pscpre.shape), _full((1, Op)),
                 _full((1, Op)), _full(gbsc.shape), _perb(mask.shape),
                 _full(A.shape), _full(Bm.shape), _full(blk['scW'].shape)],
                outs1, ospec1, sel_scratch)
        for li in range(1, len(blk['convW'])):
            W = blk['convW'][li]
            O2 = W.shape[0]
            gb = jnp.stack([blk['bng'][li - 1], blk['bnb'][li - 1]])
            Oi = y.shape[1]
            y, sp, qp = call(
                functools.partial(_mid_kernel, n2=n2),
                (y, sp, qp, gb, W),
                [_perb(y.shape), _full((1, Oi)), _full((1, Oi)),
                 _full(gb.shape), _full(W.shape)],
                (jax.ShapeDtypeStruct((B, O2, KP), f32),
                 jax.ShapeDtypeStruct((1, O2), f32),
                 jax.ShapeDtypeStruct((1, O2), f32)),
                [_perb((B, O2, KP)), _full((1, O2)), _full((1, O2))])
        gb3 = jnp.stack([blk['bng'][-1], blk['bnb'][-1]])
        gbsc = jnp.stack([blk['scg'], blk['scb']])
        state = (y, sp, qp, gb3, scpre, ssc, qsc, gbsc)
        cin = y.shape[1]

    y3, s3, q3, gb3, scpre, ssc, qsc, gbsc = state
    Op = y3.shape[1]
    w1t = params['fc1W'].T
    b1 = params['fc1b'][None, :]
    w2t = jnp.pad(params['fc2W'].T, ((0, 0), (0, 3)))
    b2 = jnp.pad(params['fc2b'], (0, 3))[None, :]
    out = call(
        functools.partial(_final_kernel, nb=nb, n2=n2),
        (y3, s3, q3, gb3, scpre, ssc, qsc, gbsc, mask, w1t, b1, w2t, b2),
        [_perb(y3.shape), _full((1, Op)), _full((1, Op)), _full(gb3.shape),
         _perb(scpre.shape), _full((1, Op)), _full((1, Op)),
         _full(gbsc.shape), _perb(mask.shape), _full(w1t.shape),
         _full(b1.shape), _full(w2t.shape), _full(b2.shape)],
        jax.ShapeDtypeStruct((B, 1, 8), f32),
        _perb((B, 1, 8)))
    return out[:, 0, :5]


# speedup vs baseline: 1.0073x; 1.0073x over previous
import functools

import jax
import jax.numpy as jnp
from jax.experimental import pallas as pl
from jax.experimental.pallas import tpu as pltpu

P = 128
KNN = 16
KP = KNN * P
EPS = 1e-5
NEG = -3.0e38


def _acc(b, ref, val):
    @pl.when(b == 0)
    def _():
        ref[...] = val

    @pl.when(b > 0)
    def _():
        ref[...] += val


def _bn_coeffs(s_full, q_full, g, bvec, n):
    tot = s_full[0]
    tot2 = q_full[0]
    m = tot / n
    v = tot2 / n - m * m
    scale = g / jnp.sqrt(v + EPS)
    shift = bvec - m * scale
    return scale[:, None], shift[:, None]


def _edge_pass(b, pts, fts, A, Bm, sc_ref, y_ref, sp_ref, qp_ref):
    xx = jnp.sum(pts * pts, axis=0, keepdims=True)
    G = jnp.dot(pts.T, pts, preferred_element_type=jnp.float32)
    pd = 2.0 * G - xx - xx.T
    colidx = jax.lax.broadcasted_iota(jnp.int32, (P, P), 1)
    rowidx = jax.lax.broadcasted_iota(jnp.int32, (P, P), 0)
    base = jnp.dot(A, fts, preferred_element_type=jnp.float32)
    Hm = jnp.dot(Bm, fts, preferred_element_type=jnp.float32)
    w = pd
    for j in range(KNN + 1):
        mrow = jnp.max(w, axis=1, keepdims=True)
        cand = jnp.where(w == mrow, colidx, jnp.int32(2 ** 30))
        amin = jnp.min(cand, axis=1, keepdims=True)
        issel = colidx == amin
        w = jnp.where(issel, NEG, w)
        if j > 0:
            selT = (rowidx == jnp.reshape(amin, (1, P))).astype(jnp.float32)
            sc_ref[:, (j - 1) * P:j * P] = selT
    gath = jnp.dot(Hm, sc_ref[...], preferred_element_type=jnp.float32)
    y = gath + jnp.concatenate([base] * KNN, axis=1)
    y_ref[0] = y
    _acc(b, sp_ref, jnp.sum(y, axis=1)[None, :])
    _acc(b, qp_ref, jnp.sum(y * y, axis=1)[None, :])


def _finish_block(y3, s3, q3, gb3, scpre, ssc, qsc, gbsc, mrow, nb, n2):
    sc3, sh3 = _bn_coeffs(s3, q3, gb3[0], gb3[1], n2)
    x = jnp.maximum(sc3 * y3 + sh3, 0.0)
    acc = x[:, :P]
    for j in range(1, KNN):
        acc = acc + x[:, j * P:(j + 1) * P]
    fmean = acc * (1.0 / KNN)
    scs, shs = _bn_coeffs(ssc, qsc, gbsc[0], gbsc[1], nb)
    sc = scs * scpre + shs
    return jnp.maximum(sc + fmean, 0.0) * mrow


def _b1_kernel(pts_ref, f_ref, m_ref, s0_ref, q0_ref, gb0_ref, A_ref,
               Bm_ref, scW_ref, y_ref, sp_ref, qp_ref, scp_ref, ssc_ref,
               qsc_ref, sc_ref, *, nb):
    b = pl.program_id(0)
    mrow = m_ref[0]
    fm = f_ref[0] * mrow
    scale, shift = _bn_coeffs(s0_ref[...], q0_ref[...], gb0_ref[0],
                              gb0_ref[1], nb)
    fts = (scale * fm + shift) * mrow
    pts = pts_ref[0] * mrow + (1.0 - mrow) * 1e9
    scpre = jnp.dot(scW_ref[...], fts, preferred_element_type=jnp.float32)
    scp_ref[0] = scpre
    _acc(b, ssc_ref, jnp.sum(scpre, axis=1)[None, :])
    _acc(b, qsc_ref, jnp.sum(scpre * scpre, axis=1)[None, :])
    _edge_pass(b, pts, fts, A_ref[...], Bm_ref[...], sc_ref, y_ref,
               sp_ref, qp_ref)


def _mid_kernel(y_ref, s_ref, q_ref, gb_ref, W_ref, yo_ref, sp_ref, qp_ref,
                *, n2):
    b = pl.program_id(0)
    scale, shift = _bn_coeffs(s_ref[...], q_ref[...], gb_ref[0], gb_ref[1],
                              n2)
    x = jnp.maximum(scale * y_ref[0] + shift, 0.0)
    y = jnp.dot(W_ref[...], x, preferred_element_type=jnp.float32)
    yo_ref[0] = y
    _acc(b, sp_ref, jnp.sum(y, axis=1)[None, :])
    _acc(b, qp_ref, jnp.sum(y * y, axis=1)[None, :])


def _trans_kernel(y3_ref, s3_ref, q3_ref, gb3_ref, scp_ref, ssc_ref,
                  qsc_ref, gbsc_ref, m_ref, A_ref, Bm_ref, scW_ref,
                  yn_ref, spn_ref, qpn_ref, scpn_ref, sscn_ref, qscn_ref,
                  sc_ref, *, nb, n2):
    b = pl.program_id(0)
    mrow = m_ref[0]
    fts = _finish_block(y3_ref[0], s3_ref[...], q3_ref[...], gb3_ref[...],
                        scp_ref[0], ssc_ref[...], qsc_ref[...],
                        gbsc_ref[...], mrow, nb, n2)
    pts = fts + (1.0 - mrow) * 1e9
    scpre = jnp.dot(scW_ref[...], fts, preferred_element_type=jnp.float32)
    scpn_ref[0] = scpre
    _acc(b, sscn_ref, jnp.sum(scpre, axis=1)[None, :])
    _acc(b, qscn_ref, jnp.sum(scpre * scpre, axis=1)[None, :])
    _edge_pass(b, pts, fts, A_ref[...], Bm_ref[...], sc_ref, yn_ref,
               spn_ref, qpn_ref)


def _final_kernel(y3_ref, s3_ref, q3_ref, gb3_ref, scp_ref, ssc_ref,
                  qsc_ref, gbsc_ref, m_ref, w1_ref, b1_ref, w2_ref, b2_ref,
                  o_ref, *, nb, n2):
    mrow = m_ref[0]
    fts = _finish_block(y3_ref[0], s3_ref[...], q3_ref[...], gb3_ref[...],
                        scp_ref[0], ssc_ref[...], qsc_ref[...],
                        gbsc_ref[...], mrow, nb, n2)
    cnt = jnp.maximum(jnp.sum(mrow), 1.0)
    pooled = (jnp.sum(fts, axis=1) / cnt)[None, :]
    h = jnp.maximum(
        jnp.dot(pooled, w1_ref[...], preferred_element_type=jnp.float32)
        + b1_ref[...], 0.0)
    o_ref[0] = (jnp.dot(h, w2_ref[...], preferred_element_type=jnp.float32)
                + b2_ref[...])


def _stats0_kernel(f_ref, m_ref, s_ref, q_ref):
    b = pl.program_id(0)
    fm = f_ref[0] * m_ref[0]
    _acc(b, s_ref, jnp.sum(fm, axis=1)[None, :])
    _acc(b, q_ref, jnp.sum(fm * fm, axis=1)[None, :])


def _full(shape):
    nd = len(shape)
    return pl.BlockSpec(shape, lambda b: (0,) * nd)


def _perb(shape):
    nd = len(shape)
    return pl.BlockSpec((1,) + shape[1:], lambda b: (b,) + (0,) * (nd - 1))


def kernel(points, features, mask, params):
    B = points.shape[0]
    C0 = features.shape[1]
    nb = float(B * P)
    n2 = float(B * P * KNN)
    f32 = jnp.float32
    sel_scratch = [pltpu.VMEM((P, KP), f32)]

    def call(body, ins, in_specs, outs, out_specs, scratch=None):
        return pl.pallas_call(
            body, grid=(B,), in_specs=in_specs, out_specs=out_specs,
            out_shape=outs, scratch_shapes=scratch or [],
            interpret=False)(*ins)

    s0, q0 = call(
        _stats0_kernel, (features, mask),
        [_perb(features.shape), _perb(mask.shape)],
        (jax.ShapeDtypeStruct((1, C0), f32),
         jax.ShapeDtypeStruct((1, C0), f32)),
        [_full((1, C0)), _full((1, C0))])

    gb0 = jnp.stack([params['bn_fts_g'], params['bn_fts_b']])

    state = None
    cin = C0
    for i, blk in enumerate(params['blocks']):
        W1 = blk['convW'][0]
        A = W1[:, :cin] - W1[:, cin:]
        Bm = W1[:, cin:]
        O1 = A.shape[0]
        outs1 = (jax.ShapeDtypeStruct((B, O1, KP), f32),
                 jax.ShapeDtypeStruct((1, O1), f32),
                 jax.ShapeDtypeStruct((1, O1), f32),
                 jax.ShapeDtypeStruct((B, O1, P), f32),
                 jax.ShapeDtypeStruct((1, O1), f32),
                 jax.ShapeDtypeStruct((1, O1), f32))
        ospec1 = [_perb((B, O1, KP)), _full((1, O1)), _full((1, O1)),
                  _perb((B, O1, P)), _full((1, O1)), _full((1, O1))]
        if i == 0:
            y, sp, qp, scpre, ssc, qsc = call(
                functools.partial(_b1_kernel, nb=nb),
                (points, features, mask, s0, q0, gb0, A, Bm, blk['scW']),
                [_perb(points.shape), _perb(features.shape),
                 _perb(mask.shape), _full((1, C0)), _full((1, C0)),
                 _full(gb0.shape), _full(A.shape), _full(Bm.shape),
                 _full(blk['scW'].shape)],
                outs1, ospec1, sel_scratch)
        else:
            y3, s3, q3, gb3, pscpre, pssc, pqsc, gbsc = state
            Op = y3.shape[1]
            y, sp, qp, scpre, ssc, qsc = call(
                functools.partial(_trans_kernel, nb=nb, n2=n2),
                (y3, s3, q3, gb3, pscpre, pssc, pqsc, gbsc, mask, A, Bm,
                 blk['scW']),
                [_perb(y3.shape), _full((1, Op)), _full((1, Op)),
                 _full(gb3.shape), _perb(pscpre.shape), _full((1, Op)),
                 _full((1, Op)), _full(gbsc.shape), _perb(mask.shape),
                 _full(A.shape), _full(Bm.shape), _full(blk['scW'].shape)],
                outs1, ospec1, sel_scratch)
        for li in range(1, len(blk['convW'])):
            W = blk['convW'][li]
            O2 = W.shape[0]
            gb = jnp.stack([blk['bng'][li - 1], blk['bnb'][li - 1]])
            Oi = y.shape[1]
            y, sp, qp = call(
                functools.partial(_mid_kernel, n2=n2),
                (y, sp, qp, gb, W),
                [_perb(y.shape), _full((1, Oi)), _full((1, Oi)),
                 _full(gb.shape), _full(W.shape)],
                (jax.ShapeDtypeStruct((B, O2, KP), f32),
                 jax.ShapeDtypeStruct((1, O2), f32),
                 jax.ShapeDtypeStruct((1, O2), f32)),
                [_perb((B, O2, KP)), _full((1, O2)), _full((1, O2))])
        gb3 = jnp.stack([blk['bng'][-1], blk['bnb'][-1]])
        gbsc = jnp.stack([blk['scg'], blk['scb']])
        state = (y, sp, qp, gb3, scpre, ssc, qsc, gbsc)
        cin = y.shape[1]

    y3, s3, q3, gb3, scpre, ssc, qsc, gbsc = state
    Op = y3.shape[1]
    w1t = params['fc1W'].T
    b1 = params['fc1b'][None, :]
    w2t = jnp.pad(params['fc2W'].T, ((0, 0), (0, 3)))
    b2 = jnp.pad(params['fc2b'], (0, 3))[None, :]
    out = call(
        functools.partial(_final_kernel, nb=nb, n2=n2),
        (y3, s3, q3, gb3, scpre, ssc, qsc, gbsc, mask, w1t, b1, w2t, b2),
        [_perb(y3.shape), _full((1, Op)), _full((1, Op)), _full(gb3.shape),
         _perb(scpre.shape), _full((1, Op)), _full((1, Op)),
         _full(gbsc.shape), _perb(mask.shape), _full(w1t.shape),
         _full(b1.shape), _full(w2t.shape), _full(b2.shape)],
        jax.ShapeDtypeStruct((B, 1, 8), f32),
        _perb((B, 1, 8)))
    return out[:, 0, :5]


# speedup vs baseline: 1.0940x; 1.0860x over previous
import functools

import jax
import jax.numpy as jnp
from jax.experimental import pallas as pl

P = 128
KNN = 16
EPS = 1e-5
NEG = -3.0e38


def _acc(b, ref, val):
    @pl.when(b == 0)
    def _():
        ref[...] = val

    @pl.when(b > 0)
    def _():
        ref[...] += val


def _bn_coeffs(s_full, q_full, g, bvec, n):
    tot = jnp.sum(s_full, axis=1)
    tot2 = jnp.sum(q_full, axis=1)
    m = tot / n
    v = tot2 / n - m * m
    scale = g / jnp.sqrt(v + EPS)
    shift = bvec - m * scale
    return scale[:, None], shift[:, None]


def _edge_pass(b, pts, fts, A, Bm, y_ref, sp_ref, qp_ref):
    xx = jnp.sum(pts * pts, axis=0, keepdims=True)
    G = jnp.dot(pts.T, pts, preferred_element_type=jnp.float32)
    pd = 2.0 * G - xx - xx.T
    colidx = jax.lax.broadcasted_iota(jnp.int32, (P, P), 1)
    base = jnp.dot(A, fts, preferred_element_type=jnp.float32)
    Hm = jnp.dot(Bm, fts, preferred_element_type=jnp.float32)
    w = pd
    ssum = None
    sq = None
    for j in range(KNN + 1):
        mrow = jnp.max(w, axis=1, keepdims=True)
        cand = jnp.where(w == mrow, colidx, jnp.int32(2 ** 30))
        amin = jnp.min(cand, axis=1, keepdims=True)
        issel = colidx == amin
        sel = issel.astype(jnp.float32)
        w = jnp.where(issel, NEG, w)
        if j > 0:
            gj = jnp.dot(Hm, sel.T, preferred_element_type=jnp.float32)
            yj = base + gj
            y_ref[0, j - 1] = yj
            ssum = yj if ssum is None else ssum + yj
            sq = yj * yj if sq is None else sq + yj * yj
    _acc(b, sp_ref, ssum)
    _acc(b, qp_ref, sq)


def _finish_block(y3, s3, q3, gb3, scpre, ssc, qsc, gbsc, mrow, nb, n2):
    sc3, sh3 = _bn_coeffs(s3, q3, gb3[0], gb3[1], n2)
    acc = None
    for j in range(KNN):
        x = jnp.maximum(sc3 * y3[j] + sh3, 0.0)
        acc = x if acc is None else acc + x
    fmean = acc * (1.0 / KNN)
    scs, shs = _bn_coeffs(ssc, qsc, gbsc[0], gbsc[1], nb)
    sc = scs * scpre + shs
    return jnp.maximum(sc + fmean, 0.0) * mrow


def _b1_kernel(pts_ref, f_ref, m_ref, s0_ref, q0_ref, gb0_ref, A_ref,
               Bm_ref, scW_ref, y_ref, sp_ref, qp_ref, scp_ref, ssc_ref,
               qsc_ref, *, nb):
    b = pl.program_id(0)
    mrow = m_ref[0]
    fm = f_ref[0] * mrow
    scale, shift = _bn_coeffs(s0_ref[...], q0_ref[...], gb0_ref[0],
                              gb0_ref[1], nb)
    fts = (scale * fm + shift) * mrow
    pts = pts_ref[0] * mrow + (1.0 - mrow) * 1e9
    scpre = jnp.dot(scW_ref[...], fts, preferred_element_type=jnp.float32)
    scp_ref[0] = scpre
    _acc(b, ssc_ref, scpre)
    _acc(b, qsc_ref, scpre * scpre)
    _edge_pass(b, pts, fts, A_ref[...], Bm_ref[...], y_ref, sp_ref, qp_ref)


def _mid_kernel(y_ref, s_ref, q_ref, gb_ref, W_ref, yo_ref, sp_ref, qp_ref,
                *, n2):
    b = pl.program_id(0)
    scale, shift = _bn_coeffs(s_ref[...], q_ref[...], gb_ref[0], gb_ref[1],
                              n2)
    W = W_ref[...]
    ssum = None
    sq = None
    for j in range(KNN):
        x = jnp.maximum(scale * y_ref[0, j] + shift, 0.0)
        yj = jnp.dot(W, x, preferred_element_type=jnp.float32)
        yo_ref[0, j] = yj
        ssum = yj if ssum is None else ssum + yj
        sq = yj * yj if sq is None else sq + yj * yj
    _acc(b, sp_ref, ssum)
    _acc(b, qp_ref, sq)


def _trans_kernel(y3_ref, s3_ref, q3_ref, gb3_ref, scp_ref, ssc_ref,
                  qsc_ref, gbsc_ref, m_ref, A_ref, Bm_ref, scW_ref,
                  yn_ref, spn_ref, qpn_ref, scpn_ref, sscn_ref, qscn_ref,
                  *, nb, n2):
    b = pl.program_id(0)
    mrow = m_ref[0]
    fts = _finish_block(y3_ref[0], s3_ref[...], q3_ref[...], gb3_ref[...],
                        scp_ref[0], ssc_ref[...], qsc_ref[...],
                        gbsc_ref[...], mrow, nb, n2)
    pts = fts + (1.0 - mrow) * 1e9
    scpre = jnp.dot(scW_ref[...], fts, preferred_element_type=jnp.float32)
    scpn_ref[0] = scpre
    _acc(b, sscn_ref, scpre)
    _acc(b, qscn_ref, scpre * scpre)
    _edge_pass(b, pts, fts, A_ref[...], Bm_ref[...], yn_ref, spn_ref,
               qpn_ref)


def _final_kernel(y3_ref, s3_ref, q3_ref, gb3_ref, scp_ref, ssc_ref,
                  qsc_ref, gbsc_ref, m_ref, w1_ref, b1_ref, w2_ref, b2_ref,
                  o_ref, *, nb, n2):
    mrow = m_ref[0]
    fts = _finish_block(y3_ref[0], s3_ref[...], q3_ref[...], gb3_ref[...],
                        scp_ref[0], ssc_ref[...], qsc_ref[...],
                        gbsc_ref[...], mrow, nb, n2)
    cnt = jnp.maximum(jnp.sum(mrow), 1.0)
    pooled = (jnp.sum(fts, axis=1) / cnt)[None, :]
    h = jnp.maximum(
        jnp.dot(pooled, w1_ref[...], preferred_element_type=jnp.float32)
        + b1_ref[...], 0.0)
    o_ref[0] = (jnp.dot(h, w2_ref[...], preferred_element_type=jnp.float32)
                + b2_ref[...])


def _stats0_kernel(f_ref, m_ref, s_ref, q_ref):
    b = pl.program_id(0)
    fm = f_ref[0] * m_ref[0]
    _acc(b, s_ref, fm)
    _acc(b, q_ref, fm * fm)


def _full(shape):
    nd = len(shape)
    return pl.BlockSpec(shape, lambda b: (0,) * nd)


def _perb(shape):
    nd = len(shape)
    return pl.BlockSpec((1,) + shape[1:], lambda b: (b,) + (0,) * (nd - 1))


def kernel(points, features, mask, params):
    B = points.shape[0]
    C0 = features.shape[1]
    nb = float(B * P)
    n2 = float(B * P * KNN)
    f32 = jnp.float32

    def call(body, ins, in_specs, outs, out_specs):
        return pl.pallas_call(
            body, grid=(B,), in_specs=in_specs, out_specs=out_specs,
            out_shape=outs, interpret=False)(*ins)

    s0, q0 = call(
        _stats0_kernel, (features, mask),
        [_perb(features.shape), _perb(mask.shape)],
        (jax.ShapeDtypeStruct((C0, P), f32),
         jax.ShapeDtypeStruct((C0, P), f32)),
        [_full((C0, P)), _full((C0, P))])

    gb0 = jnp.stack([params['bn_fts_g'], params['bn_fts_b']])

    state = None
    cin = C0
    for i, blk in enumerate(params['blocks']):
        W1 = blk['convW'][0]
        A = W1[:, :cin] - W1[:, cin:]
        Bm = W1[:, cin:]
        O1 = A.shape[0]
        outs1 = (jax.ShapeDtypeStruct((B, KNN, O1, P), f32),
                 jax.ShapeDtypeStruct((O1, P), f32),
                 jax.ShapeDtypeStruct((O1, P), f32),
                 jax.ShapeDtypeStruct((B, O1, P), f32),
                 jax.ShapeDtypeStruct((O1, P), f32),
                 jax.ShapeDtypeStruct((O1, P), f32))
        ospec1 = [_perb((B, KNN, O1, P)), _full((O1, P)), _full((O1, P)),
                  _perb((B, O1, P)), _full((O1, P)), _full((O1, P))]
        if i == 0:
            y, sp, qp, scpre, ssc, qsc = call(
                functools.partial(_b1_kernel, nb=nb),
                (points, features, mask, s0, q0, gb0, A, Bm, blk['scW']),
                [_perb(points.shape), _perb(features.shape),
                 _perb(mask.shape), _full((C0, P)), _full((C0, P)),
                 _full(gb0.shape), _full(A.shape), _full(Bm.shape),
                 _full(blk['scW'].shape)],
                outs1, ospec1)
        else:
            y3, s3, q3, gb3, pscpre, pssc, pqsc, gbsc = state
            Op = y3.shape[2]
            y, sp, qp, scpre, ssc, qsc = call(
                functools.partial(_trans_kernel, nb=nb, n2=n2),
                (y3, s3, q3, gb3, pscpre, pssc, pqsc, gbsc, mask, A, Bm,
                 blk['scW']),
                [_perb(y3.shape), _full((Op, P)), _full((Op, P)),
                 _full(gb3.shape), _perb(pscpre.shape), _full((Op, P)),
                 _full((Op, P)), _full(gbsc.shape), _perb(mask.shape),
                 _full(A.shape), _full(Bm.shape), _full(blk['scW'].shape)],
                outs1, ospec1)
        for li in range(1, len(blk['convW'])):
            W = blk['convW'][li]
            O2 = W.shape[0]
            gb = jnp.stack([blk['bng'][li - 1], blk['bnb'][li - 1]])
            Oi = y.shape[2]
            y, sp, qp = call(
                functools.partial(_mid_kernel, n2=n2),
                (y, sp, qp, gb, W),
                [_perb(y.shape), _full((Oi, P)), _full((Oi, P)),
                 _full(gb.shape), _full(W.shape)],
                (jax.ShapeDtypeStruct((B, KNN, O2, P), f32),
                 jax.ShapeDtypeStruct((O2, P), f32),
                 jax.ShapeDtypeStruct((O2, P), f32)),
                [_perb((B, KNN, O2, P)), _full((O2, P)), _full((O2, P))])
        gb3 = jnp.stack([blk['bng'][-1], blk['bnb'][-1]])
        gbsc = jnp.stack([blk['scg'], blk['scb']])
        state = (y, sp, qp, gb3, scpre, ssc, qsc, gbsc)
        cin = y.shape[2]

    y3, s3, q3, gb3, scpre, ssc, qsc, gbsc = state
    Op = y3.shape[2]
    w1t = params['fc1W'].T
    b1 = params['fc1b'][None, :]
    w2t = jnp.pad(params['fc2W'].T, ((0, 0), (0, 3)))
    b2 = jnp.pad(params['fc2b'], (0, 3))[None, :]
    out = call(
        functools.partial(_final_kernel, nb=nb, n2=n2),
        (y3, s3, q3, gb3, scpre, ssc, qsc, gbsc, mask, w1t, b1, w2t, b2),
        [_perb(y3.shape), _full((Op, P)), _full((Op, P)), _full(gb3.shape),
         _perb(scpre.shape), _full((Op, P)), _full((Op, P)),
         _full(gbsc.shape), _perb(mask.shape), _full(w1t.shape),
         _full(b1.shape), _full(w2t.shape), _full(b2.shape)],
        jax.ShapeDtypeStruct((B, 1, 8), f32),
        _perb((B, 1, 8)))
    return out[:, 0, :5]
